# baseline (speedup 1.0000x reference)
import functools

import jax
import jax.numpy as jnp
from jax import lax
from jax.experimental import pallas as pl
from jax.experimental.pallas import tpu as pltpu
from jax.experimental.pallas import tpu_sc as plsc

_INTERPRET = False


def _softmax_body(x_ref, wr_ref, br_ref, gt_ref, pt_ref, pb_ref):
    logits = lax.dot_general(
        wr_ref[...], x_ref[...], (((0,), (1,)), ((), ())),
        preferred_element_type=jnp.float32)
    logits = logits + br_ref[...] + gt_ref[...]
    m = jnp.max(logits, axis=0, keepdims=True)
    e = jnp.exp(logits - m)
    p = e / jnp.sum(e, axis=0, keepdims=True)
    pt_ref[...] = p
    pb_ref[...] = lax.bitcast_convert_type(p, jnp.int32)


def _sc_topk_body(k_sel, E, bs, pt_hbm, pb_hbm, w_hbm, vals, bits, contrib,
                  red, shared):
    core = lax.axis_index("c")
    sub = lax.axis_index("s")
    seg = bs // E
    i32 = jnp.int32
    U = 16

    def lane_sum(acc):
        total = acc[0]
        for j in range(1, 16):
            total = total + acc[j]
        return total.astype(i32)

    def count_pass(pred):
        def body(i, acc):
            base = pl.multiple_of(i * (16 * U), 16 * U)
            for j in range(U):
                v = bits[pl.ds(base + 16 * j, 16)]
                idx = lax.iota(i32, 16) + (i * (16 * U) + 16 * j)
                acc = acc + jnp.where(pred(v, idx), 1.0, 0.0)
            return acc
        acc = lax.fori_loop(0, bs // (16 * U), body,
                            jnp.zeros((16,), jnp.float32))
        return lane_sum(acc)

    @pl.when((core == 0) & (sub < E))
    def _expert():
        pltpu.sync_copy(pt_hbm.at[sub], vals)
        pltpu.sync_copy(pb_hbm.at[sub], bits)
        def bitloop(b, T):
            lo = i32(28) - 2 * b
            chi = T | (i32(1) << (lo + 1))
            c3 = chi | (i32(1) << lo)
            c1 = T | (i32(1) << lo)
            def body(i, carry):
                a1, a2, a3 = carry
                base = pl.multiple_of(i * (16 * U), 16 * U)
                for j in range(U):
                    v = bits[pl.ds(base + 16 * j, 16)]
                    a1 = a1 + jnp.where(v >= c1, 1.0, 0.0)
                    a2 = a2 + jnp.where(v >= chi, 1.0, 0.0)
                    a3 = a3 + jnp.where(v >= c3, 1.0, 0.0)
                return a1, a2, a3
            z = jnp.zeros((16,), jnp.float32)
            a1, a2, a3 = lax.fori_loop(0, bs // (16 * U), body, (z, z, z))
            n1, n2, n3 = lane_sum(a1), lane_sum(a2), lane_sum(a3)
            return jnp.where(n3 >= k_sel, c3,
                             jnp.where(n2 >= k_sel, chi,
                                       jnp.where(n1 >= k_sel, c1, T)))
        T = lax.fori_loop(0, 15, bitloop, i32(0))
        cnt_gt = count_pass(lambda v, idx: v > T)
        r = k_sel - cnt_gt
        cnt_eq = count_pass(lambda v, idx: v == T)

        def tie_search(_):
            def bitloop2(b, M):
                cand = M | (i32(1) << (i32(12) - b))
                cnt = count_pass(lambda v, idx: (v == T) & (idx < cand))
                return jnp.where(cnt < r, cand, M)
            return lax.fori_loop(0, 13, bitloop2, i32(0))
        M = lax.cond(cnt_eq == r, lambda _: i32(bs), tie_search, 0)

        def emit(i, carry):
            base = pl.multiple_of(i * (16 * U), 16 * U)
            for j in range(U):
                off = base + 16 * j
                xv = vals[pl.ds(off, 16)]
                v = bits[pl.ds(off, 16)]
                idx = lax.iota(i32, 16) + (i * (16 * U) + 16 * j)
                inc = (v > T) | ((v == T) & (idx <= M))
                contrib[pl.ds(off, 16)] = jnp.where(inc, xv, 0.0)
            return carry
        lax.fori_loop(0, bs // (16 * U), emit, i32(0))
        pltpu.sync_copy(contrib, shared.at[sub])

    plsc.subcore_barrier()

    @pl.when((core == 0) & (sub < E))
    def _reduce():
        base = pl.multiple_of(sub * seg, seg)
        for e in range(E):
            pltpu.sync_copy(shared.at[e, pl.ds(base, seg)], red.at[e])
        def body(i, carry):
            off = pl.multiple_of(i * 16, 16)
            total = red[0, pl.ds(off, 16)]
            for e in range(1, E):
                total = total + red[e, pl.ds(off, 16)]
            contrib[pl.ds(off, 16)] = total
            return carry
        lax.fori_loop(0, seg // 16, body, i32(0))
        pltpu.sync_copy(contrib.at[pl.ds(0, seg)], w_hbm.at[pl.ds(base, seg)])


def _ffn_body(x_ref, w1_ref, w2_ref, w3_ref, wv_ref, o_ref):
    xb = x_ref[...].astype(jnp.bfloat16)
    a = jnp.dot(xb, w1_ref[...], preferred_element_type=jnp.float32)
    b = jnp.dot(xb, w2_ref[...], preferred_element_type=jnp.float32)
    h = (a * lax.logistic(a) * b).astype(jnp.bfloat16)
    o = jnp.dot(h, w3_ref[...], preferred_element_type=jnp.float32)
    o_ref[...] = o * wv_ref[...]


def kernel(x, W_router, b_router, w1, w2, w3, gumbel_noise):
    B, S, H = x.shape
    bs = B * S
    E = W_router.shape[1]
    k_sel = min(bs // E, bs)
    FF = w1.shape[1]
    xf = x.reshape(bs, H)

    pt, pt_bits = pl.pallas_call(
        _softmax_body,
        out_shape=[jax.ShapeDtypeStruct((E, bs), jnp.float32),
                   jax.ShapeDtypeStruct((E, bs), jnp.int32)],
        interpret=_INTERPRET,
    )(xf, W_router, b_router.reshape(E, 1), gumbel_noise.T)

    sc_topk = functools.partial(
        pl.kernel,
        out_type=jax.ShapeDtypeStruct((bs,), jnp.float32),
        mesh=plsc.VectorSubcoreMesh(core_axis_name="c", subcore_axis_name="s"),
        scratch_types=[
            pltpu.VMEM((bs,), jnp.float32),
            pltpu.VMEM((bs,), jnp.int32),
            pltpu.VMEM((bs,), jnp.float32),
            pltpu.VMEM((E, bs // E), jnp.float32),
            pltpu.VMEM_SHARED((E, bs), jnp.float32),
        ],
    )(functools.partial(_sc_topk_body, k_sel, E, bs))
    wv = sc_topk(pt, pt_bits).reshape(bs, 1)

    BM = 512
    grid = (bs // BM,)
    y = pl.pallas_call(
        _ffn_body,
        grid=grid,
        in_specs=[
            pl.BlockSpec((BM, H), lambda i: (i, 0)),
            pl.BlockSpec((H, FF), lambda i: (0, 0)),
            pl.BlockSpec((H, FF), lambda i: (0, 0)),
            pl.BlockSpec((FF, H), lambda i: (0, 0)),
            pl.BlockSpec((BM, 1), lambda i: (i, 0)),
        ],
        out_specs=pl.BlockSpec((BM, H), lambda i: (i, 0)),
        out_shape=jax.ShapeDtypeStruct((bs, H), jnp.float32),
        interpret=_INTERPRET,
    )(xf, w1.astype(jnp.bfloat16), w2.astype(jnp.bfloat16),
      w3.astype(jnp.bfloat16), wv)

    return y.reshape(B, S, H)


# speedup vs baseline: 1.0553x; 1.0553x over previous
import functools

import jax
import jax.numpy as jnp
from jax import lax
from jax.experimental import pallas as pl
from jax.experimental.pallas import tpu as pltpu
from jax.experimental.pallas import tpu_sc as plsc

_INTERPRET = False


def _softmax_body(x_ref, wr_ref, br_ref, gt_ref, pt_ref, pb_ref):
    logits = lax.dot_general(
        wr_ref[...], x_ref[...], (((0,), (1,)), ((), ())),
        preferred_element_type=jnp.float32)
    logits = logits + br_ref[...] + gt_ref[...]
    m = jnp.max(logits, axis=0, keepdims=True)
    e = jnp.exp(logits - m)
    p = e / jnp.sum(e, axis=0, keepdims=True)
    pt_ref[...] = p
    pb_ref[...] = lax.bitcast_convert_type(p, jnp.int32)


def _sc_topk_body(k_sel, E, bs, pt_hbm, pb_hbm, w_hbm, vals, bits, contrib,
                  red, shared):
    core = lax.axis_index("c")
    sub = lax.axis_index("s")
    seg = bs // E
    i32 = jnp.int32
    U = 8

    def lane_sum(acc):
        total = acc[0]
        for j in range(1, 16):
            total = total + acc[j]
        return total.astype(i32)

    def count_pass(pred):
        def body(i, acc):
            base = pl.multiple_of(i * (16 * U), 16 * U)
            for j in range(U):
                v = bits[pl.ds(base + 16 * j, 16)]
                idx = lax.iota(i32, 16) + (i * (16 * U) + 16 * j)
                acc = acc + jnp.where(pred(v, idx), 1.0, 0.0)
            return acc
        acc = lax.fori_loop(0, bs // (16 * U), body,
                            jnp.zeros((16,), jnp.float32))
        return lane_sum(acc)

    @pl.when((core == 0) & (sub < E))
    def _expert():
        pltpu.sync_copy(pt_hbm.at[sub], vals)
        pltpu.sync_copy(pb_hbm.at[sub], bits)
        def bitloop(b, T):
            lo = i32(28) - 2 * b
            chi = T | (i32(1) << (lo + 1))
            c3 = chi | (i32(1) << lo)
            c1 = T | (i32(1) << lo)
            def body(i, carry):
                a1, a2, a3 = carry
                base = pl.multiple_of(i * (16 * U), 16 * U)
                for j in range(U):
                    v = bits[pl.ds(base + 16 * j, 16)]
                    a1 = a1 + jnp.where(v >= c1, 1.0, 0.0)
                    a2 = a2 + jnp.where(v >= chi, 1.0, 0.0)
                    a3 = a3 + jnp.where(v >= c3, 1.0, 0.0)
                return a1, a2, a3
            z = jnp.zeros((16,), jnp.float32)
            a1, a2, a3 = lax.fori_loop(0, bs // (16 * U), body, (z, z, z))
            n1, n2, n3 = lane_sum(a1), lane_sum(a2), lane_sum(a3)
            return jnp.where(n3 >= k_sel, c3,
                             jnp.where(n2 >= k_sel, chi,
                                       jnp.where(n1 >= k_sel, c1, T)))
        T = lax.fori_loop(0, 15, bitloop, i32(0))
        cnt_gt = count_pass(lambda v, idx: v > T)
        r = k_sel - cnt_gt
        cnt_eq = count_pass(lambda v, idx: v == T)

        def tie_search(_):
            def bitloop2(b, M):
                cand = M | (i32(1) << (i32(12) - b))
                cnt = count_pass(lambda v, idx: (v == T) & (idx < cand))
                return jnp.where(cnt < r, cand, M)
            return lax.fori_loop(0, 13, bitloop2, i32(0))
        M = lax.cond(cnt_eq == r, lambda _: i32(bs), tie_search, 0)

        def emit(i, carry):
            base = pl.multiple_of(i * (16 * U), 16 * U)
            for j in range(U):
                off = base + 16 * j
                xv = vals[pl.ds(off, 16)]
                v = bits[pl.ds(off, 16)]
                idx = lax.iota(i32, 16) + (i * (16 * U) + 16 * j)
                inc = (v > T) | ((v == T) & (idx <= M))
                contrib[pl.ds(off, 16)] = jnp.where(inc, xv, 0.0)
            return carry
        lax.fori_loop(0, bs // (16 * U), emit, i32(0))
        pltpu.sync_copy(contrib, shared.at[sub])

    plsc.subcore_barrier()

    @pl.when((core == 0) & (sub < E))
    def _reduce():
        base = pl.multiple_of(sub * seg, seg)
        for e in range(E):
            pltpu.sync_copy(shared.at[e, pl.ds(base, seg)], red.at[e])
        def body(i, carry):
            off = pl.multiple_of(i * 16, 16)
            total = red[0, pl.ds(off, 16)]
            for e in range(1, E):
                total = total + red[e, pl.ds(off, 16)]
            contrib[pl.ds(off, 16)] = total
            return carry
        lax.fori_loop(0, seg // 16, body, i32(0))
        pltpu.sync_copy(contrib.at[pl.ds(0, seg)], w_hbm.at[pl.ds(base, seg)])


def _ffn_body(x_ref, w1_ref, w2_ref, w3_ref, wv_ref, o_ref):
    xb = x_ref[...].astype(jnp.bfloat16)
    a = jnp.dot(xb, w1_ref[...], preferred_element_type=jnp.float32)
    b = jnp.dot(xb, w2_ref[...], preferred_element_type=jnp.float32)
    h = (a * lax.logistic(a) * b).astype(jnp.bfloat16)
    o = jnp.dot(h, w3_ref[...], preferred_element_type=jnp.float32)
    o_ref[...] = o * wv_ref[...]


def kernel(x, W_router, b_router, w1, w2, w3, gumbel_noise):
    B, S, H = x.shape
    bs = B * S
    E = W_router.shape[1]
    k_sel = min(bs // E, bs)
    FF = w1.shape[1]
    xf = x.reshape(bs, H)

    pt, pt_bits = pl.pallas_call(
        _softmax_body,
        out_shape=[jax.ShapeDtypeStruct((E, bs), jnp.float32),
                   jax.ShapeDtypeStruct((E, bs), jnp.int32)],
        interpret=_INTERPRET,
    )(xf, W_router, b_router.reshape(E, 1), gumbel_noise.T)

    sc_topk = functools.partial(
        pl.kernel,
        out_type=jax.ShapeDtypeStruct((bs,), jnp.float32),
        mesh=plsc.VectorSubcoreMesh(core_axis_name="c", subcore_axis_name="s"),
        scratch_types=[
            pltpu.VMEM((bs,), jnp.float32),
            pltpu.VMEM((bs,), jnp.int32),
            pltpu.VMEM((bs,), jnp.float32),
            pltpu.VMEM((E, bs // E), jnp.float32),
            pltpu.VMEM_SHARED((E, bs), jnp.float32),
        ],
    )(functools.partial(_sc_topk_body, k_sel, E, bs))
    wv = sc_topk(pt, pt_bits).reshape(bs, 1)

    BM = 512
    grid = (bs // BM,)
    y = pl.pallas_call(
        _ffn_body,
        grid=grid,
        in_specs=[
            pl.BlockSpec((BM, H), lambda i: (i, 0)),
            pl.BlockSpec((H, FF), lambda i: (0, 0)),
            pl.BlockSpec((H, FF), lambda i: (0, 0)),
            pl.BlockSpec((FF, H), lambda i: (0, 0)),
            pl.BlockSpec((BM, 1), lambda i: (i, 0)),
        ],
        out_specs=pl.BlockSpec((BM, H), lambda i: (i, 0)),
        out_shape=jax.ShapeDtypeStruct((bs, H), jnp.float32),
        interpret=_INTERPRET,
    )(xf, w1.astype(jnp.bfloat16), w2.astype(jnp.bfloat16),
      w3.astype(jnp.bfloat16), wv)

    return y.reshape(B, S, H)
